# baseline (speedup 1.0000x reference)
import functools

import jax
import jax.numpy as jnp
from jax import lax
from jax.experimental import pallas as pl
from jax.experimental.pallas import tpu as pltpu
from jax.experimental.pallas import tpu_sc as plsc

N = 10000
F_IN = 128
HID = 64
HW = 128
N_GRAPHS = 64
N_CLASSES = 10

NP = 10240
R = 1280
NBLK = NP // R

NC = 2
NS = 16
NW = NC * NS
K = 128
RPT = NP // NS

_MESH = dict(core_axis_name="c", subcore_axis_name="s")


def _sc_hist(dstp, ones_rows, zinit, nch):
    mesh = plsc.VectorSubcoreMesh(**_MESH)

    @functools.partial(
        pl.kernel,
        out_type=jax.ShapeDtypeStruct((NC, NP, HW), jnp.float32),
        mesh=mesh,
        scratch_types=[
            pltpu.VMEM((nch, K), jnp.int32),
            pltpu.VMEM((K, HW), jnp.float32),
            pltpu.VMEM_SHARED((NP, HW), jnp.float32),
        ],
    )
    def k(dst_hbm, ones_hbm, z_hbm, out_hbm, dstv, onesv, acc):
        c = lax.axis_index("c")
        s = lax.axis_index("s")
        wid = s * NC + c
        base = s * RPT
        pltpu.sync_copy(z_hbm.at[pl.ds(base, RPT)], acc.at[pl.ds(base, RPT)])
        pltpu.sync_copy(ones_hbm, onesv)
        pltpu.sync_copy(dst_hbm.at[wid], dstv)
        plsc.subcore_barrier()

        def step(j, carry):
            pltpu.sync_copy(onesv, acc.at[dstv.at[j]], add=True)
            return carry

        lax.fori_loop(0, nch, step, 0)
        plsc.subcore_barrier()
        pltpu.sync_copy(acc.at[pl.ds(base, RPT)],
                        out_hbm.at[c, pl.ds(base, RPT)])

    return k(dstp, ones_rows, zinit)


def _sc_msg(eidx, y, zinit, nch0, nch1):
    mesh = plsc.VectorSubcoreMesh(**_MESH)

    @functools.partial(
        pl.kernel,
        out_type=jax.ShapeDtypeStruct((NC, NP, HW), jnp.float32),
        mesh=mesh,
        scratch_types=[
            pltpu.VMEM((4, 2, K), jnp.int32),
            pltpu.VMEM((2, K, HW), jnp.float32),
            pltpu.VMEM_SHARED((NP, HW), jnp.float32),
            pltpu.SemaphoreType.DMA((2,)),
            pltpu.SemaphoreType.DMA((2,)),
            pltpu.SemaphoreType.DMA((4,)),
        ],
    )
    def k(eidx_hbm, y_hbm, z_hbm, out_hbm, idxr, rows, acc, gsem, ssem, isem):
        c = lax.axis_index("c")
        s = lax.axis_index("s")
        base = s * RPT
        start = lax.select(c == 0, s * nch0, NS * nch0 + s * nch1)
        cnt = lax.select(c == 0, nch0, nch1)
        pltpu.sync_copy(z_hbm.at[pl.ds(base, RPT)], acc.at[pl.ds(base, RPT)])
        plsc.subcore_barrier()

        def chunk_step(j, b, do_next, do_prev_wait, do_fetch):
            p = b % 2
            r = b % 4
            r1 = (b + 1) % 4
            rf = (b + 3) % 4
            pltpu.make_async_copy(y_hbm.at[idxr.at[r, 0]], rows.at[p],
                                  gsem.at[p]).wait()
            if do_prev_wait:
                pltpu.make_async_copy(rows.at[1 - p], acc.at[idxr.at[r, 1]],
                                      ssem.at[1 - p]).wait()
            pltpu.async_copy(rows.at[p], acc.at[idxr.at[r, 1]], ssem.at[p],
                             add=True)
            if do_next:
                pltpu.make_async_copy(eidx_hbm.at[0], idxr.at[r1],
                                      isem.at[r1]).wait()
                pltpu.async_copy(y_hbm.at[idxr.at[r1, 0]], rows.at[1 - p],
                                 gsem.at[1 - p])
                if do_fetch:
                    pltpu.async_copy(eidx_hbm.at[start + j + 3], idxr.at[rf],
                                     isem.at[rf])

        for r_ in range(4):
            pltpu.async_copy(eidx_hbm.at[start + r_], idxr.at[r_],
                             isem.at[r_])
        pltpu.make_async_copy(eidx_hbm.at[0], idxr.at[0], isem.at[0]).wait()
        pltpu.async_copy(y_hbm.at[idxr.at[0, 0]], rows.at[0], gsem.at[0])

        chunk_step(0, 0, True, False, False)
        for b in range(1, 4):
            chunk_step(b, b, True, True, True)

        def macro(m, carry):
            for b in range(4):
                chunk_step(4 * m + b, b, True, True, True)
            return carry

        lax.fori_loop(1, cnt // 4 - 1, macro, 0)

        j0 = cnt - 4
        chunk_step(j0, 0, True, True, True)
        chunk_step(j0 + 1, 1, True, True, False)
        chunk_step(j0 + 2, 2, True, True, False)
        chunk_step(j0 + 3, 3, False, True, False)
        pltpu.make_async_copy(rows.at[1], acc.at[idxr.at[0, 1]],
                              ssem.at[1]).wait()

        plsc.subcore_barrier()
        pltpu.sync_copy(acc.at[pl.ds(base, RPT)],
                        out_hbm.at[c, pl.ds(base, RPT)])

    return k(eidx, y, zinit)


def _tc_matmul(x_pad, W1p):
    def body(x_ref, w_ref, o_ref):
        o_ref[...] = jnp.dot(x_ref[...], w_ref[...],
                             preferred_element_type=jnp.float32)

    return pl.pallas_call(
        body,
        grid=(NBLK,),
        in_specs=[
            pl.BlockSpec((R, F_IN), lambda i: (i, 0)),
            pl.BlockSpec((F_IN, HW), lambda i: (0, 0)),
        ],
        out_specs=pl.BlockSpec((R, HW), lambda i: (i, 0)),
        out_shape=jax.ShapeDtypeStruct((NP, HW), jnp.float32),
    )(x_pad, W1p)


def _tc_scale1(xw, hist):
    def body(xw_ref, h_ref, y_ref, dis_ref):
        i = pl.program_id(0)
        h = h_ref[...]
        deg = h[0] + h[1] + 1.0
        dis = lax.rsqrt(deg)
        row = lax.broadcasted_iota(jnp.int32, (R, HW), 0) + i * R
        dis = jnp.where(row < N, dis, 0.0)
        y_ref[...] = dis * xw_ref[...]
        dis_ref[...] = dis

    return pl.pallas_call(
        body,
        grid=(NBLK,),
        in_specs=[
            pl.BlockSpec((R, HW), lambda i: (i, 0)),
            pl.BlockSpec((NC, R, HW), lambda i: (0, i, 0)),
        ],
        out_specs=[
            pl.BlockSpec((R, HW), lambda i: (i, 0)),
            pl.BlockSpec((R, HW), lambda i: (i, 0)),
        ],
        out_shape=[
            jax.ShapeDtypeStruct((NP, HW), jnp.float32),
            jax.ShapeDtypeStruct((NP, HW), jnp.float32),
        ],
    )(xw, hist)


def _tc_layer2(acc1, y1, dis, b1r, W2p):
    def body(a_ref, y1_ref, d_ref, b_ref, w_ref, y2_ref):
        a = a_ref[...]
        d = d_ref[...]
        o = d * (a[0] + a[1] + y1_ref[...]) + b_ref[...]
        h = jnp.maximum(o, 0.0)
        y2_ref[...] = d * jnp.dot(h, w_ref[...],
                                  preferred_element_type=jnp.float32)

    return pl.pallas_call(
        body,
        grid=(NBLK,),
        in_specs=[
            pl.BlockSpec((NC, R, HW), lambda i: (0, i, 0)),
            pl.BlockSpec((R, HW), lambda i: (i, 0)),
            pl.BlockSpec((R, HW), lambda i: (i, 0)),
            pl.BlockSpec((1, HW), lambda i: (0, 0)),
            pl.BlockSpec((HW, HW), lambda i: (0, 0)),
        ],
        out_specs=pl.BlockSpec((R, HW), lambda i: (i, 0)),
        out_shape=jax.ShapeDtypeStruct((NP, HW), jnp.float32),
    )(acc1, y1, dis, b1r, W2p)


def _tc_head(acc2, y2, dis, b2r, batch2d, fcW1p, fb1r, fcW2p, fb2r):
    def body(a_ref, y2_ref, d_ref, b_ref, bt_ref, w1_ref, c1_ref, w2_ref,
             c2_ref, out_ref, gsum, cnt):
        i = pl.program_id(0)

        @pl.when(i == 0)
        def _():
            gsum[...] = jnp.zeros((N_GRAPHS, HW), jnp.float32)
            cnt[...] = jnp.zeros((N_GRAPHS, 1), jnp.float32)

        a = a_ref[...]
        d = d_ref[...]
        o = d * (a[0] + a[1] + y2_ref[...]) + b_ref[...]
        h2 = jnp.maximum(o, 0.0)
        bt = bt_ref[...]
        gid = lax.broadcasted_iota(jnp.int32, (N_GRAPHS, 1), 0)
        oh = (bt == gid).astype(jnp.float32)
        gsum[...] += jnp.dot(oh, h2, preferred_element_type=jnp.float32)
        cnt[...] += jnp.dot(oh, jnp.ones((R, 1), jnp.float32),
                            preferred_element_type=jnp.float32)

        @pl.when(i == NBLK - 1)
        def _():
            g = gsum[...] / jnp.maximum(cnt[...], 1.0)
            z = jnp.maximum(
                jnp.dot(g, w1_ref[...], preferred_element_type=jnp.float32)
                + c1_ref[...], 0.0)
            out_ref[...] = (jnp.dot(z, w2_ref[...],
                                    preferred_element_type=jnp.float32)
                            + c2_ref[...])

    return pl.pallas_call(
        body,
        grid=(NBLK,),
        in_specs=[
            pl.BlockSpec((NC, R, HW), lambda i: (0, i, 0)),
            pl.BlockSpec((R, HW), lambda i: (i, 0)),
            pl.BlockSpec((R, HW), lambda i: (i, 0)),
            pl.BlockSpec((1, HW), lambda i: (0, 0)),
            pl.BlockSpec((1, R), lambda i: (0, i)),
            pl.BlockSpec((HW, HW), lambda i: (0, 0)),
            pl.BlockSpec((1, HW), lambda i: (0, 0)),
            pl.BlockSpec((HW, N_CLASSES), lambda i: (0, 0)),
            pl.BlockSpec((1, N_CLASSES), lambda i: (0, 0)),
        ],
        out_specs=pl.BlockSpec((N_GRAPHS, N_CLASSES), lambda i: (0, 0)),
        out_shape=jax.ShapeDtypeStruct((N_GRAPHS, N_CLASSES), jnp.float32),
        scratch_shapes=[
            pltpu.VMEM((N_GRAPHS, HW), jnp.float32),
            pltpu.VMEM((N_GRAPHS, 1), jnp.float32),
        ],
        compiler_params=pltpu.CompilerParams(
            dimension_semantics=("arbitrary",)),
    )(acc2, y2, dis, b2r, batch2d, fcW1p, fb1r, fcW2p, fb2r)


def kernel(x, edge_index, batch, W1, b1, W2, b2, fcW1, fcb1, fcW2, fcb2):
    n, f_in = x.shape
    e = edge_index.shape[1]
    nch = -(-e // (NW * K))
    nch = -(-nch // 4) * 4
    ep = NW * nch * K
    ncht = NW * nch
    nch1 = (3 * ncht // (4 * NS)) // 4 * 4
    nch0 = ncht // NS - nch1

    x_pad = jnp.zeros((NP, f_in), jnp.float32).at[:n].set(x)
    pad = jnp.full((ep - e,), n, jnp.int32)
    srcp = jnp.concatenate([edge_index[0], pad]).reshape(NW, nch, K)
    dstp = jnp.concatenate([edge_index[1], pad]).reshape(NW, nch, K)
    eidx = jnp.stack([srcp.reshape(ncht, K), dstp.reshape(ncht, K)],
                     axis=1)
    batch2d = jnp.concatenate(
        [batch, jnp.full((NP - n,), N_GRAPHS, jnp.int32)]).reshape(1, NP)
    ones_rows = jnp.ones((K, HW), jnp.float32)
    zinit = jnp.zeros((NP, HW), jnp.float32)
    W1p = jnp.zeros((f_in, HW), jnp.float32).at[:, :HID].set(W1)
    W2p = jnp.zeros((HW, HW), jnp.float32).at[:HID, :HID].set(W2)
    fcW1p = jnp.zeros((HW, HW), jnp.float32).at[:HID, :HID].set(fcW1)
    fcW2p = jnp.zeros((HW, N_CLASSES), jnp.float32).at[:HID].set(fcW2)
    b1r = jnp.zeros((1, HW), jnp.float32).at[0, :HID].set(b1)
    b2r = jnp.zeros((1, HW), jnp.float32).at[0, :HID].set(b2)
    fb1r = jnp.zeros((1, HW), jnp.float32).at[0, :HID].set(fcb1)
    fb2r = fcb2.reshape(1, N_CLASSES)

    hist = _sc_hist(dstp, ones_rows, zinit, nch)
    xw = _tc_matmul(x_pad, W1p)
    y1, dis = _tc_scale1(xw, hist)
    acc1 = _sc_msg(eidx, y1, zinit, nch0, nch1)
    y2 = _tc_layer2(acc1, y1, dis, b1r, W2p)
    acc2 = _sc_msg(eidx, y2, zinit, nch0, nch1)
    return _tc_head(acc2, y2, dis, b2r, batch2d, fcW1p, fb1r, fcW2p, fb2r)


# speedup vs baseline: 1.4999x; 1.4999x over previous
import functools

import jax
import jax.numpy as jnp
from jax import lax
from jax.experimental import pallas as pl
from jax.experimental.pallas import tpu as pltpu
from jax.experimental.pallas import tpu_sc as plsc

N = 10000
F_IN = 128
HID = 64
HW = 128
N_GRAPHS = 64
N_CLASSES = 10

NP = 10240
R = 1280
NBLK = NP // R

NC = 2
NS = 16
NW = NC * NS
K = 128
RPT = NP // NS

_MESH = dict(core_axis_name="c", subcore_axis_name="s")


def _sc_hist(dstp, ones_rows, zinit, nch):
    mesh = plsc.VectorSubcoreMesh(**_MESH)

    @functools.partial(
        pl.kernel,
        out_type=jax.ShapeDtypeStruct((NC, NP, HW), jnp.float32),
        mesh=mesh,
        scratch_types=[
            pltpu.VMEM((nch, K), jnp.int32),
            pltpu.VMEM((K, HW), jnp.float32),
            pltpu.VMEM_SHARED((NP, HW), jnp.float32),
        ],
    )
    def k(dst_hbm, ones_hbm, z_hbm, out_hbm, dstv, onesv, acc):
        c = lax.axis_index("c")
        s = lax.axis_index("s")
        wid = s * NC + c
        base = s * RPT
        pltpu.sync_copy(z_hbm.at[pl.ds(base, RPT)], acc.at[pl.ds(base, RPT)])
        pltpu.sync_copy(ones_hbm, onesv)
        pltpu.sync_copy(dst_hbm.at[wid], dstv)
        plsc.subcore_barrier()

        def step(j, carry):
            pltpu.sync_copy(onesv, acc.at[dstv.at[j]], add=True)
            return carry

        lax.fori_loop(0, nch, step, 0)
        plsc.subcore_barrier()
        pltpu.sync_copy(acc.at[pl.ds(base, RPT)],
                        out_hbm.at[c, pl.ds(base, RPT)])

    return k(dstp, ones_rows, zinit)


def _sc_msg(srcp, dstp, y, zinit, nch):
    mesh = plsc.VectorSubcoreMesh(**_MESH)

    @functools.partial(
        pl.kernel,
        out_type=jax.ShapeDtypeStruct((NC, NP, HW), jnp.float32),
        mesh=mesh,
        scratch_types=[
            pltpu.VMEM((nch, K), jnp.int32),
            pltpu.VMEM((nch, K), jnp.int32),
            pltpu.VMEM((K, HW), jnp.float32),
            pltpu.VMEM_SHARED((NP, HW), jnp.float32),
            pltpu.SemaphoreType.DMA,
        ],
    )
    def k(src_hbm, dst_hbm, y_hbm, z_hbm, out_hbm, srcv, dstv, rows, acc,
          sem):
        c = lax.axis_index("c")
        s = lax.axis_index("s")
        wid = s * NC + c
        base = s * RPT
        pltpu.sync_copy(z_hbm.at[pl.ds(base, RPT)], acc.at[pl.ds(base, RPT)])
        pltpu.sync_copy(src_hbm.at[wid], srcv)
        pltpu.sync_copy(dst_hbm.at[wid], dstv)
        plsc.subcore_barrier()

        def step(j, carry):
            pltpu.async_copy(y_hbm.at[srcv.at[j]], rows, sem).wait()
            pltpu.sync_copy(rows, acc.at[dstv.at[j]], add=True)
            return carry

        lax.fori_loop(0, nch, step, 0)
        plsc.subcore_barrier()
        pltpu.sync_copy(acc.at[pl.ds(base, RPT)],
                        out_hbm.at[c, pl.ds(base, RPT)])

    return k(srcp, dstp, y, zinit)


def _tc_matmul(x_pad, W1p):
    def body(x_ref, w_ref, o_ref):
        o_ref[...] = jnp.dot(x_ref[...], w_ref[...],
                             preferred_element_type=jnp.float32)

    return pl.pallas_call(
        body,
        grid=(NBLK,),
        in_specs=[
            pl.BlockSpec((R, F_IN), lambda i: (i, 0)),
            pl.BlockSpec((F_IN, HW), lambda i: (0, 0)),
        ],
        out_specs=pl.BlockSpec((R, HW), lambda i: (i, 0)),
        out_shape=jax.ShapeDtypeStruct((NP, HW), jnp.float32),
    )(x_pad, W1p)


def _tc_scale1(xw, hist):
    def body(xw_ref, h_ref, y_ref, dis_ref):
        i = pl.program_id(0)
        h = h_ref[...]
        deg = h[0] + h[1] + 1.0
        dis = lax.rsqrt(deg)
        row = lax.broadcasted_iota(jnp.int32, (R, HW), 0) + i * R
        dis = jnp.where(row < N, dis, 0.0)
        y_ref[...] = dis * xw_ref[...]
        dis_ref[...] = dis

    return pl.pallas_call(
        body,
        grid=(NBLK,),
        in_specs=[
            pl.BlockSpec((R, HW), lambda i: (i, 0)),
            pl.BlockSpec((NC, R, HW), lambda i: (0, i, 0)),
        ],
        out_specs=[
            pl.BlockSpec((R, HW), lambda i: (i, 0)),
            pl.BlockSpec((R, HW), lambda i: (i, 0)),
        ],
        out_shape=[
            jax.ShapeDtypeStruct((NP, HW), jnp.float32),
            jax.ShapeDtypeStruct((NP, HW), jnp.float32),
        ],
    )(xw, hist)


def _tc_layer2(acc1, y1, dis, b1r, W2p):
    def body(a_ref, y1_ref, d_ref, b_ref, w_ref, y2_ref):
        a = a_ref[...]
        d = d_ref[...]
        o = d * (a[0] + a[1] + y1_ref[...]) + b_ref[...]
        h = jnp.maximum(o, 0.0)
        y2_ref[...] = d * jnp.dot(h, w_ref[...],
                                  preferred_element_type=jnp.float32)

    return pl.pallas_call(
        body,
        grid=(NBLK,),
        in_specs=[
            pl.BlockSpec((NC, R, HW), lambda i: (0, i, 0)),
            pl.BlockSpec((R, HW), lambda i: (i, 0)),
            pl.BlockSpec((R, HW), lambda i: (i, 0)),
            pl.BlockSpec((1, HW), lambda i: (0, 0)),
            pl.BlockSpec((HW, HW), lambda i: (0, 0)),
        ],
        out_specs=pl.BlockSpec((R, HW), lambda i: (i, 0)),
        out_shape=jax.ShapeDtypeStruct((NP, HW), jnp.float32),
    )(acc1, y1, dis, b1r, W2p)


def _tc_head(acc2, y2, dis, b2r, batch2d, fcW1p, fb1r, fcW2p, fb2r):
    def body(a_ref, y2_ref, d_ref, b_ref, bt_ref, w1_ref, c1_ref, w2_ref,
             c2_ref, out_ref, gsum, cnt):
        i = pl.program_id(0)

        @pl.when(i == 0)
        def _():
            gsum[...] = jnp.zeros((N_GRAPHS, HW), jnp.float32)
            cnt[...] = jnp.zeros((N_GRAPHS, 1), jnp.float32)

        a = a_ref[...]
        d = d_ref[...]
        o = d * (a[0] + a[1] + y2_ref[...]) + b_ref[...]
        h2 = jnp.maximum(o, 0.0)
        bt = bt_ref[...]
        gid = lax.broadcasted_iota(jnp.int32, (N_GRAPHS, 1), 0)
        oh = (bt == gid).astype(jnp.float32)
        gsum[...] += jnp.dot(oh, h2, preferred_element_type=jnp.float32)
        cnt[...] += jnp.dot(oh, jnp.ones((R, 1), jnp.float32),
                            preferred_element_type=jnp.float32)

        @pl.when(i == NBLK - 1)
        def _():
            g = gsum[...] / jnp.maximum(cnt[...], 1.0)
            z = jnp.maximum(
                jnp.dot(g, w1_ref[...], preferred_element_type=jnp.float32)
                + c1_ref[...], 0.0)
            out_ref[...] = (jnp.dot(z, w2_ref[...],
                                    preferred_element_type=jnp.float32)
                            + c2_ref[...])

    return pl.pallas_call(
        body,
        grid=(NBLK,),
        in_specs=[
            pl.BlockSpec((NC, R, HW), lambda i: (0, i, 0)),
            pl.BlockSpec((R, HW), lambda i: (i, 0)),
            pl.BlockSpec((R, HW), lambda i: (i, 0)),
            pl.BlockSpec((1, HW), lambda i: (0, 0)),
            pl.BlockSpec((1, R), lambda i: (0, i)),
            pl.BlockSpec((HW, HW), lambda i: (0, 0)),
            pl.BlockSpec((1, HW), lambda i: (0, 0)),
            pl.BlockSpec((HW, N_CLASSES), lambda i: (0, 0)),
            pl.BlockSpec((1, N_CLASSES), lambda i: (0, 0)),
        ],
        out_specs=pl.BlockSpec((N_GRAPHS, N_CLASSES), lambda i: (0, 0)),
        out_shape=jax.ShapeDtypeStruct((N_GRAPHS, N_CLASSES), jnp.float32),
        scratch_shapes=[
            pltpu.VMEM((N_GRAPHS, HW), jnp.float32),
            pltpu.VMEM((N_GRAPHS, 1), jnp.float32),
        ],
        compiler_params=pltpu.CompilerParams(
            dimension_semantics=("arbitrary",)),
    )(acc2, y2, dis, b2r, batch2d, fcW1p, fb1r, fcW2p, fb2r)


def kernel(x, edge_index, batch, W1, b1, W2, b2, fcW1, fcb1, fcW2, fcb2):
    n, f_in = x.shape
    e = edge_index.shape[1]
    nch = -(-e // (NW * K))
    ep = NW * nch * K

    x_pad = jnp.zeros((NP, f_in), jnp.float32).at[:n].set(x)
    pad = jnp.full((ep - e,), n, jnp.int32)
    srcp = jnp.concatenate([edge_index[0], pad]).reshape(NW, nch, K)
    dstp = jnp.concatenate([edge_index[1], pad]).reshape(NW, nch, K)
    batch2d = jnp.concatenate(
        [batch, jnp.full((NP - n,), N_GRAPHS, jnp.int32)]).reshape(1, NP)
    ones_rows = jnp.ones((K, HW), jnp.float32)
    zinit = jnp.zeros((NP, HW), jnp.float32)
    W1p = jnp.zeros((f_in, HW), jnp.float32).at[:, :HID].set(W1)
    W2p = jnp.zeros((HW, HW), jnp.float32).at[:HID, :HID].set(W2)
    fcW1p = jnp.zeros((HW, HW), jnp.float32).at[:HID, :HID].set(fcW1)
    fcW2p = jnp.zeros((HW, N_CLASSES), jnp.float32).at[:HID].set(fcW2)
    b1r = jnp.zeros((1, HW), jnp.float32).at[0, :HID].set(b1)
    b2r = jnp.zeros((1, HW), jnp.float32).at[0, :HID].set(b2)
    fb1r = jnp.zeros((1, HW), jnp.float32).at[0, :HID].set(fcb1)
    fb2r = fcb2.reshape(1, N_CLASSES)

    hist = _sc_hist(dstp, ones_rows, zinit, nch)
    xw = _tc_matmul(x_pad, W1p)
    y1, dis = _tc_scale1(xw, hist)
    acc1 = _sc_msg(srcp, dstp, y1, zinit, nch)
    y2 = _tc_layer2(acc1, y1, dis, b1r, W2p)
    acc2 = _sc_msg(srcp, dstp, y2, zinit, nch)
    return _tc_head(acc2, y2, dis, b2r, batch2d, fcW1p, fb1r, fcW2p, fb2r)


# speedup vs baseline: 1.6697x; 1.1132x over previous
import functools

import jax
import jax.numpy as jnp
from jax import lax
from jax.experimental import pallas as pl
from jax.experimental.pallas import tpu as pltpu
from jax.experimental.pallas import tpu_sc as plsc

N = 10000
F_IN = 128
HID = 64
HW = 128
N_GRAPHS = 64
N_CLASSES = 10

NP = 10240
R = 1280
NBLK = NP // R

NC = 2
NS = 16
NW = NC * NS
K = 128
KH = 64
RPT = NP // NS

_MESH = dict(core_axis_name="c", subcore_axis_name="s")


def _sc_hist(dstp, ones_rows, zinit, nch):
    mesh = plsc.VectorSubcoreMesh(**_MESH)

    @functools.partial(
        pl.kernel,
        out_type=jax.ShapeDtypeStruct((NC, NP, HW), jnp.float32),
        mesh=mesh,
        scratch_types=[
            pltpu.VMEM((nch, K), jnp.int32),
            pltpu.VMEM((K, HW), jnp.float32),
            pltpu.VMEM_SHARED((NP, HW), jnp.float32),
        ],
    )
    def k(dst_hbm, ones_hbm, z_hbm, out_hbm, dstv, onesv, acc):
        c = lax.axis_index("c")
        s = lax.axis_index("s")
        wid = s * NC + c
        base = s * RPT
        pltpu.sync_copy(z_hbm.at[pl.ds(base, RPT)], acc.at[pl.ds(base, RPT)])
        pltpu.sync_copy(ones_hbm, onesv)
        pltpu.sync_copy(dst_hbm.at[wid], dstv)
        plsc.subcore_barrier()

        def step(j, carry):
            pltpu.sync_copy(onesv, acc.at[dstv.at[j]], add=True)
            return carry

        lax.fori_loop(0, nch, step, 0)
        plsc.subcore_barrier()
        pltpu.sync_copy(acc.at[pl.ds(base, RPT)],
                        out_hbm.at[c, pl.ds(base, RPT)])

    return k(dstp, ones_rows, zinit)


def _sc_msg(srcp, dstp, y, zinit, nch):
    mesh = plsc.VectorSubcoreMesh(**_MESH)
    tot = 2 * nch

    @functools.partial(
        pl.kernel,
        out_type=jax.ShapeDtypeStruct((NC, NP, HW), jnp.float32),
        mesh=mesh,
        scratch_types=[
            pltpu.VMEM((nch * K,), jnp.int32),
            pltpu.VMEM((nch, 2, KH), jnp.int32),
            pltpu.VMEM((2, KH, HW), jnp.float32),
            pltpu.VMEM_SHARED((NP, HW), jnp.float32),
            pltpu.SemaphoreType.DMA((2,)),
        ],
    )
    def k(src_hbm, dst_hbm, y_hbm, z_hbm, out_hbm, srcv, dstv, rows, acc,
          gsem):
        c = lax.axis_index("c")
        s = lax.axis_index("s")
        wid = s * NC + c
        base = s * RPT
        pltpu.sync_copy(z_hbm.at[pl.ds(base, RPT)], acc.at[pl.ds(base, RPT)])
        pltpu.sync_copy(src_hbm.at[wid], srcv)
        pltpu.sync_copy(dst_hbm.at[wid], dstv)
        plsc.subcore_barrier()

        pltpu.async_copy(y_hbm.at[srcv.at[pl.ds(0, KH)]], rows.at[0],
                         gsem.at[0])

        def step(u, carry):
            p = lax.rem(u, 2)
            pltpu.make_async_copy(y_hbm.at[srcv.at[pl.ds(0, KH)]],
                                  rows.at[p], gsem.at[p]).wait()

            @pl.when(u + 1 < tot)
            def _():
                pltpu.async_copy(
                    y_hbm.at[srcv.at[pl.ds((u + 1) * KH, KH)]],
                    rows.at[1 - p], gsem.at[1 - p])

            pltpu.sync_copy(rows.at[p],
                            acc.at[dstv.at[lax.div(u, 2), lax.rem(u, 2)]],
                            add=True)
            return carry

        lax.fori_loop(0, tot, step, 0)
        plsc.subcore_barrier()
        pltpu.sync_copy(acc.at[pl.ds(base, RPT)],
                        out_hbm.at[c, pl.ds(base, RPT)])

    return k(srcp, dstp, y, zinit)


def _tc_matmul(x_pad, W1p):
    def body(x_ref, w_ref, o_ref):
        o_ref[...] = jnp.dot(x_ref[...], w_ref[...],
                             preferred_element_type=jnp.float32)

    return pl.pallas_call(
        body,
        grid=(NBLK,),
        in_specs=[
            pl.BlockSpec((R, F_IN), lambda i: (i, 0)),
            pl.BlockSpec((F_IN, HW), lambda i: (0, 0)),
        ],
        out_specs=pl.BlockSpec((R, HW), lambda i: (i, 0)),
        out_shape=jax.ShapeDtypeStruct((NP, HW), jnp.float32),
    )(x_pad, W1p)


def _tc_scale1(xw, hist):
    def body(xw_ref, h_ref, y_ref, dis_ref):
        i = pl.program_id(0)
        h = h_ref[...]
        deg = h[0] + h[1] + 1.0
        dis = lax.rsqrt(deg)
        row = lax.broadcasted_iota(jnp.int32, (R, HW), 0) + i * R
        dis = jnp.where(row < N, dis, 0.0)
        y_ref[...] = dis * xw_ref[...]
        dis_ref[...] = dis

    return pl.pallas_call(
        body,
        grid=(NBLK,),
        in_specs=[
            pl.BlockSpec((R, HW), lambda i: (i, 0)),
            pl.BlockSpec((NC, R, HW), lambda i: (0, i, 0)),
        ],
        out_specs=[
            pl.BlockSpec((R, HW), lambda i: (i, 0)),
            pl.BlockSpec((R, HW), lambda i: (i, 0)),
        ],
        out_shape=[
            jax.ShapeDtypeStruct((NP, HW), jnp.float32),
            jax.ShapeDtypeStruct((NP, HW), jnp.float32),
        ],
    )(xw, hist)


def _tc_layer2(acc1, y1, dis, b1r, W2p):
    def body(a_ref, y1_ref, d_ref, b_ref, w_ref, y2_ref):
        a = a_ref[...]
        d = d_ref[...]
        o = d * (a[0] + a[1] + y1_ref[...]) + b_ref[...]
        h = jnp.maximum(o, 0.0)
        y2_ref[...] = d * jnp.dot(h, w_ref[...],
                                  preferred_element_type=jnp.float32)

    return pl.pallas_call(
        body,
        grid=(NBLK,),
        in_specs=[
            pl.BlockSpec((NC, R, HW), lambda i: (0, i, 0)),
            pl.BlockSpec((R, HW), lambda i: (i, 0)),
            pl.BlockSpec((R, HW), lambda i: (i, 0)),
            pl.BlockSpec((1, HW), lambda i: (0, 0)),
            pl.BlockSpec((HW, HW), lambda i: (0, 0)),
        ],
        out_specs=pl.BlockSpec((R, HW), lambda i: (i, 0)),
        out_shape=jax.ShapeDtypeStruct((NP, HW), jnp.float32),
    )(acc1, y1, dis, b1r, W2p)


def _tc_head(acc2, y2, dis, b2r, batch2d, fcW1p, fb1r, fcW2p, fb2r):
    def body(a_ref, y2_ref, d_ref, b_ref, bt_ref, w1_ref, c1_ref, w2_ref,
             c2_ref, out_ref, gsum, cnt):
        i = pl.program_id(0)

        @pl.when(i == 0)
        def _():
            gsum[...] = jnp.zeros((N_GRAPHS, HW), jnp.float32)
            cnt[...] = jnp.zeros((N_GRAPHS, 1), jnp.float32)

        a = a_ref[...]
        d = d_ref[...]
        o = d * (a[0] + a[1] + y2_ref[...]) + b_ref[...]
        h2 = jnp.maximum(o, 0.0)
        bt = bt_ref[...]
        gid = lax.broadcasted_iota(jnp.int32, (N_GRAPHS, 1), 0)
        oh = (bt == gid).astype(jnp.float32)
        gsum[...] += jnp.dot(oh, h2, preferred_element_type=jnp.float32)
        cnt[...] += jnp.dot(oh, jnp.ones((R, 1), jnp.float32),
                            preferred_element_type=jnp.float32)

        @pl.when(i == NBLK - 1)
        def _():
            g = gsum[...] / jnp.maximum(cnt[...], 1.0)
            z = jnp.maximum(
                jnp.dot(g, w1_ref[...], preferred_element_type=jnp.float32)
                + c1_ref[...], 0.0)
            out_ref[...] = (jnp.dot(z, w2_ref[...],
                                    preferred_element_type=jnp.float32)
                            + c2_ref[...])

    return pl.pallas_call(
        body,
        grid=(NBLK,),
        in_specs=[
            pl.BlockSpec((NC, R, HW), lambda i: (0, i, 0)),
            pl.BlockSpec((R, HW), lambda i: (i, 0)),
            pl.BlockSpec((R, HW), lambda i: (i, 0)),
            pl.BlockSpec((1, HW), lambda i: (0, 0)),
            pl.BlockSpec((1, R), lambda i: (0, i)),
            pl.BlockSpec((HW, HW), lambda i: (0, 0)),
            pl.BlockSpec((1, HW), lambda i: (0, 0)),
            pl.BlockSpec((HW, N_CLASSES), lambda i: (0, 0)),
            pl.BlockSpec((1, N_CLASSES), lambda i: (0, 0)),
        ],
        out_specs=pl.BlockSpec((N_GRAPHS, N_CLASSES), lambda i: (0, 0)),
        out_shape=jax.ShapeDtypeStruct((N_GRAPHS, N_CLASSES), jnp.float32),
        scratch_shapes=[
            pltpu.VMEM((N_GRAPHS, HW), jnp.float32),
            pltpu.VMEM((N_GRAPHS, 1), jnp.float32),
        ],
        compiler_params=pltpu.CompilerParams(
            dimension_semantics=("arbitrary",)),
    )(acc2, y2, dis, b2r, batch2d, fcW1p, fb1r, fcW2p, fb2r)


def kernel(x, edge_index, batch, W1, b1, W2, b2, fcW1, fcb1, fcW2, fcb2):
    n, f_in = x.shape
    e = edge_index.shape[1]
    nch = -(-e // (NW * K))
    ep = NW * nch * K

    x_pad = jnp.zeros((NP, f_in), jnp.float32).at[:n].set(x)
    pad = jnp.full((ep - e,), n, jnp.int32)
    srcp = jnp.concatenate([edge_index[0], pad]).reshape(NW, nch, K)
    dstp = jnp.concatenate([edge_index[1], pad]).reshape(NW, nch, K)
    batch2d = jnp.concatenate(
        [batch, jnp.full((NP - n,), N_GRAPHS, jnp.int32)]).reshape(1, NP)
    ones_rows = jnp.ones((K, HW), jnp.float32)
    zinit = jnp.zeros((NP, HW), jnp.float32)
    W1p = jnp.zeros((f_in, HW), jnp.float32).at[:, :HID].set(W1)
    W2p = jnp.zeros((HW, HW), jnp.float32).at[:HID, :HID].set(W2)
    fcW1p = jnp.zeros((HW, HW), jnp.float32).at[:HID, :HID].set(fcW1)
    fcW2p = jnp.zeros((HW, N_CLASSES), jnp.float32).at[:HID].set(fcW2)
    b1r = jnp.zeros((1, HW), jnp.float32).at[0, :HID].set(b1)
    b2r = jnp.zeros((1, HW), jnp.float32).at[0, :HID].set(b2)
    fb1r = jnp.zeros((1, HW), jnp.float32).at[0, :HID].set(fcb1)
    fb2r = fcb2.reshape(1, N_CLASSES)

    srcp_m = srcp.reshape(NW, nch * K)
    dstp_m = dstp.reshape(NW, nch, 2, KH)
    hist = _sc_hist(dstp, ones_rows, zinit, nch)
    xw = _tc_matmul(x_pad, W1p)
    y1, dis = _tc_scale1(xw, hist)
    acc1 = _sc_msg(srcp_m, dstp_m, y1, zinit, nch)
    y2 = _tc_layer2(acc1, y1, dis, b1r, W2p)
    acc2 = _sc_msg(srcp_m, dstp_m, y2, zinit, nch)
    return _tc_head(acc2, y2, dis, b2r, batch2d, fcW1p, fb1r, fcW2p, fb2r)
